# baseline (speedup 1.0000x reference)
import jax
import jax.numpy as jnp
from jax import lax
from jax.experimental import pallas as pl
from jax.experimental.pallas import tpu as pltpu
from jax.experimental.pallas import tpu_sc as plsc

NNODE = 10000
DDIM = 128
NEDGE = 320000
NLAYER = 3

NCORE = 2
NSUB = 16
DH = DDIM // NCORE
CH = 64
NB = 5
NCHUNK = 320
NBLK = NCHUNK // NB
EPT = NCHUNK * CH
EPAD = NSUB * EPT
PADROWS = 8
RPW = 640
NPAD2 = NSUB * RPW
EWC = 40


def _bcast16(ref, idx):
    return plsc.load_gather(ref, [jnp.full((16,), idx, jnp.int32)])


def _sc_body(fr_hbm, to_hbm, emb_hbm, out_hbm,
             s_hbm, r_sh, deg_sh,
             fr_v, to_v, buf0, buf1, buf2, buf3, buf4, ew_r, ew_a, sbuf,
             dis_v, dis2_v, ones_v,
             gsem0, gsem1, gsem2, gsem3, gsem4,
             ssem0, ssem1, ssem2, ssem3, ssem4):
    bufs = (buf0, buf1, buf2, buf3, buf4)
    gsems = (gsem0, gsem1, gsem2, gsem3, gsem4)
    ssems = (ssem0, ssem1, ssem2, ssem3, ssem4)
    c = lax.axis_index("c")
    s = lax.axis_index("s")
    base = s * RPW
    nrows = jnp.minimum(RPW, jnp.maximum(0, NNODE - base))
    n_ew = nrows // EWC

    pltpu.sync_copy(fr_hbm.at[s], fr_v)
    pltpu.sync_copy(to_hbm.at[s], to_v)

    sbase = c * NPAD2

    def bias_fr(j, _):
        for cv in range(CH // 16):
            sl = pl.ds(cv * 16, 16)
            fr_v[j, sl] = fr_v[j, sl] + sbase
        return 0
    lax.fori_loop(0, NCHUNK, bias_fr, 0)

    def fill_ones(k, _):
        ones_v[pl.ds(k * 16, 16)] = jnp.ones((16,), jnp.float32)
        return 0
    lax.fori_loop(0, CH // 16, fill_ones, 0)

    def zfill(i, _):
        for cv in range(4):
            sbuf[i, pl.ds(cv * 16, 16)] = jnp.zeros((16,), jnp.float32)
        return 0
    lax.fori_loop(0, EWC, zfill, 0)

    def zero_deg(k, _):
        pltpu.sync_copy(sbuf.at[0], deg_sh.at[pl.ds(base + k * DH, DH)])
        return 0
    lax.fori_loop(0, RPW // DH, zero_deg, 0)

    def zero_r(j, _):
        pltpu.sync_copy(sbuf, r_sh.at[pl.ds(base + j * EWC, EWC)])
        return 0
    lax.fori_loop(0, RPW // EWC, zero_r, 0)

    def zero_spad(j, _):
        pltpu.sync_copy(sbuf, s_hbm.at[pl.ds(sbase + base + j * EWC, EWC)])
        return 0
    lax.fori_loop(n_ew, RPW // EWC, zero_spad, 0)

    plsc.subcore_barrier()

    def deg_blk(b, _):
        descs = []
        for u in range(NB):
            descs.append(pltpu.async_copy(
                ones_v, deg_sh.at[to_v.at[b * NB + u]], ssems[u], add=True))
        for d in descs:
            d.wait()
        return 0
    lax.fori_loop(0, NBLK, deg_blk, 0)

    plsc.subcore_barrier()

    pltpu.sync_copy(deg_sh.at[pl.ds(base, RPW)], dis_v)

    def newton(k, _):
        sl = pl.ds(k * 16, 16)
        d = dis_v[sl]
        i = lax.bitcast_convert_type(d, jnp.int32)
        i = jnp.int32(0x5F3759DF) - lax.shift_right_logical(i, 1)
        y = lax.bitcast_convert_type(i, jnp.float32)
        for _ in range(3):
            y = y * (1.5 - 0.5 * d * y * y)
        y = jnp.where(d > 0.0, y, 0.0)
        dis_v[sl] = y
        dis2_v[sl] = y * y
        return 0
    lax.fori_loop(0, RPW // 16, newton, 0)

    def s0_blk(j, _):
        r0 = base + j * EWC
        pltpu.sync_copy(emb_hbm.at[c, pl.ds(r0, EWC)], ew_r)

        def row(i, _):
            b = _bcast16(dis_v, j * EWC + i)
            for cv in range(4):
                sl = pl.ds(cv * 16, 16)
                sbuf[i, sl] = b * ew_r[i, sl]
            return 0
        lax.fori_loop(0, EWC, row, 0)
        pltpu.sync_copy(sbuf, s_hbm.at[pl.ds(sbase + r0, EWC)])
        return 0
    lax.fori_loop(0, n_ew, s0_blk, 0)

    plsc.subcore_barrier()

    for layer in range(NLAYER):
        last = layer == NLAYER - 1

        def edge_blk(b, _):
            gds = []
            for u in range(NB):
                gds.append(pltpu.async_copy(
                    s_hbm.at[fr_v.at[b * NB + u]], bufs[u], gsems[u]))
            sds = []
            for u in range(NB):
                gds[u].wait()
                sds.append(pltpu.async_copy(
                    bufs[u], r_sh.at[to_v.at[b * NB + u]], ssems[u],
                    add=True))
            for d in sds:
                d.wait()
            return 0
        lax.fori_loop(0, NBLK, edge_blk, 0)

        plsc.subcore_barrier()

        def ew_blk(j, _):
            r0 = base + j * EWC
            pltpu.sync_copy(r_sh.at[pl.ds(r0, EWC)], ew_r)
            if layer == 0:
                pltpu.sync_copy(ew_r, out_hbm.at[c, pl.ds(r0, EWC)])
            else:
                pltpu.sync_copy(out_hbm.at[c, pl.ds(r0, EWC)], ew_a)
            if last:
                pltpu.sync_copy(emb_hbm.at[c, pl.ds(r0, EWC)],
                                buf0.at[pl.ds(0, EWC)])

            def row(i, _):
                row_l = j * EWC + i
                for cv in range(4):
                    sl = pl.ds(cv * 16, 16)
                    rv = ew_r[i, sl]
                    if last:
                        b = _bcast16(dis_v, row_l)
                        acc = ew_a[i, sl] + rv
                        sbuf[i, sl] = 0.25 * buf0[i, sl] + 0.25 * b * acc
                    else:
                        b2 = _bcast16(dis2_v, row_l)
                        if layer > 0:
                            ew_a[i, sl] = ew_a[i, sl] + rv
                        sbuf[i, sl] = b2 * rv
                return 0
            lax.fori_loop(0, EWC, row, 0)

            if last:
                pltpu.sync_copy(sbuf, out_hbm.at[c, pl.ds(r0, EWC)])
            else:
                if layer > 0:
                    pltpu.sync_copy(ew_a, out_hbm.at[c, pl.ds(r0, EWC)])
                pltpu.sync_copy(sbuf, s_hbm.at[pl.ds(sbase + r0, EWC)])

                def zf(i, _):
                    for cv in range(4):
                        sbuf[i, pl.ds(cv * 16, 16)] = (
                            jnp.zeros((16,), jnp.float32))
                    return 0
                lax.fori_loop(0, EWC, zf, 0)
                pltpu.sync_copy(sbuf, r_sh.at[pl.ds(r0, EWC)])
            return 0
        lax.fori_loop(0, n_ew, ew_blk, 0)

        if not last:
            plsc.subcore_barrier()


_sc_call = pl.kernel(
    _sc_body,
    out_type=jax.ShapeDtypeStruct((NCORE, NNODE, DH), jnp.float32),
    mesh=plsc.VectorSubcoreMesh(
        core_axis_name="c", subcore_axis_name="s",
        num_cores=NCORE, num_subcores=NSUB),
    scratch_types=[
        pltpu.HBM((NCORE * NPAD2, DH), jnp.float32),
        pltpu.VMEM_SHARED((NPAD2, DH), jnp.float32),
        pltpu.VMEM_SHARED((NPAD2,), jnp.float32),
        pltpu.VMEM((NCHUNK, CH), jnp.int32),
        pltpu.VMEM((NCHUNK, CH), jnp.int32),
        pltpu.VMEM((CH, DH), jnp.float32),
        pltpu.VMEM((CH, DH), jnp.float32),
        pltpu.VMEM((CH, DH), jnp.float32),
        pltpu.VMEM((CH, DH), jnp.float32),
        pltpu.VMEM((CH, DH), jnp.float32),
        pltpu.VMEM((EWC, DH), jnp.float32),
        pltpu.VMEM((EWC, DH), jnp.float32),
        pltpu.VMEM((EWC, DH), jnp.float32),
        pltpu.VMEM((RPW,), jnp.float32),
        pltpu.VMEM((RPW,), jnp.float32),
        pltpu.VMEM((CH,), jnp.float32),
        pltpu.SemaphoreType.DMA,
        pltpu.SemaphoreType.DMA,
        pltpu.SemaphoreType.DMA,
        pltpu.SemaphoreType.DMA,
        pltpu.SemaphoreType.DMA,
        pltpu.SemaphoreType.DMA,
        pltpu.SemaphoreType.DMA,
        pltpu.SemaphoreType.DMA,
        pltpu.SemaphoreType.DMA,
        pltpu.SemaphoreType.DMA,
    ],
    compiler_params=pltpu.CompilerParams(
        needs_layout_passes=False, use_tc_tiling_on_sc=False),
)


@jax.jit
def kernel(edge_index, edge_attrs, emb_weight):
    del edge_attrs
    npad = EPAD - NEDGE
    padidx = (jnp.arange(npad, dtype=jnp.int32) % PADROWS) + NNODE
    fr3 = jnp.concatenate([edge_index[0], padidx]).reshape(NSUB, NCHUNK, CH)
    to3 = jnp.concatenate([edge_index[1], padidx]).reshape(NSUB, NCHUNK, CH)
    emb2 = emb_weight.reshape(NNODE, NCORE, DH).transpose(1, 0, 2)
    out2 = _sc_call(fr3, to3, emb2)
    out = out2.transpose(1, 0, 2).reshape(NNODE, DDIM)
    return (emb_weight, out)


# speedup vs baseline: 1.1300x; 1.1300x over previous
import jax
import jax.numpy as jnp
from jax import lax
from jax.experimental import pallas as pl
from jax.experimental.pallas import tpu as pltpu
from jax.experimental.pallas import tpu_sc as plsc

NNODE = 10000
DDIM = 128
NEDGE = 320000
NLAYER = 3

NCORE = 2
NSUB = 16
DH = DDIM // NCORE
CH = 64
NB = 5
IG = 20
NGROUP = 16
NCHUNK = 320
EPT = NCHUNK * CH
EPAD = NSUB * EPT
PADROWS = 8
RPW = 640
NPAD2 = NSUB * RPW
EWC = 40


def _bcast16(ref, idx):
    return plsc.load_gather(ref, [jnp.full((16,), idx, jnp.int32)])


def _sc_body(fr_hbm, to_hbm, emb_hbm, out_hbm,
             s_sh, r_sh, deg_sh,
             fr_g, to_g, buf0, buf1, buf2, buf3, buf4, ew_r, ew_a, sbuf,
             dis_v, dis2_v, ones_v,
             gsem0, gsem1, gsem2, gsem3, gsem4,
             ssem0, ssem1, ssem2, ssem3, ssem4):
    bufs = (buf0, buf1, buf2, buf3, buf4)
    gsems = (gsem0, gsem1, gsem2, gsem3, gsem4)
    ssems = (ssem0, ssem1, ssem2, ssem3, ssem4)
    c = lax.axis_index("c")
    s = lax.axis_index("s")
    base = s * RPW
    nrows = jnp.minimum(RPW, jnp.maximum(0, NNODE - base))
    n_ew = nrows // EWC

    def fill_ones(k, _):
        ones_v[pl.ds(k * 16, 16)] = jnp.ones((16,), jnp.float32)
        return 0
    lax.fori_loop(0, CH // 16, fill_ones, 0)

    def zfill(i, _):
        for cv in range(4):
            sbuf[i, pl.ds(cv * 16, 16)] = jnp.zeros((16,), jnp.float32)
        return 0
    lax.fori_loop(0, EWC, zfill, 0)

    def zero_deg(k, _):
        pltpu.sync_copy(sbuf.at[0], deg_sh.at[pl.ds(base + k * DH, DH)])
        return 0
    lax.fori_loop(0, RPW // DH, zero_deg, 0)

    def zero_r(j, _):
        pltpu.sync_copy(sbuf, r_sh.at[pl.ds(base + j * EWC, EWC)])
        return 0
    lax.fori_loop(0, RPW // EWC, zero_r, 0)

    def zero_spad(j, _):
        pltpu.sync_copy(sbuf, s_sh.at[pl.ds(base + j * EWC, EWC)])
        return 0
    lax.fori_loop(n_ew, RPW // EWC, zero_spad, 0)

    plsc.subcore_barrier()

    def deg_grp(g, _):
        pltpu.sync_copy(to_hbm.at[s, pl.ds(g * IG, IG)], to_g)

        def deg_blk(b, _):
            descs = []
            for u in range(NB):
                descs.append(pltpu.async_copy(
                    ones_v, deg_sh.at[to_g.at[b * NB + u]], ssems[u],
                    add=True))
            for d in descs:
                d.wait()
            return 0
        lax.fori_loop(0, IG // NB, deg_blk, 0)
        return 0
    lax.fori_loop(0, NGROUP, deg_grp, 0)

    plsc.subcore_barrier()

    pltpu.sync_copy(deg_sh.at[pl.ds(base, RPW)], dis_v)

    def newton(k, _):
        sl = pl.ds(k * 16, 16)
        d = dis_v[sl]
        i = lax.bitcast_convert_type(d, jnp.int32)
        i = jnp.int32(0x5F3759DF) - lax.shift_right_logical(i, 1)
        y = lax.bitcast_convert_type(i, jnp.float32)
        for _ in range(3):
            y = y * (1.5 - 0.5 * d * y * y)
        y = jnp.where(d > 0.0, y, 0.0)
        dis_v[sl] = y
        dis2_v[sl] = y * y
        return 0
    lax.fori_loop(0, RPW // 16, newton, 0)

    def s0_blk(j, _):
        r0 = base + j * EWC
        pltpu.sync_copy(emb_hbm.at[c, pl.ds(r0, EWC)], ew_r)

        def row(i, _):
            b = _bcast16(dis_v, j * EWC + i)
            for cv in range(4):
                sl = pl.ds(cv * 16, 16)
                sbuf[i, sl] = b * ew_r[i, sl]
            return 0
        lax.fori_loop(0, EWC, row, 0)
        pltpu.sync_copy(sbuf, s_sh.at[pl.ds(r0, EWC)])
        return 0
    lax.fori_loop(0, n_ew, s0_blk, 0)

    plsc.subcore_barrier()

    for layer in range(NLAYER):
        last = layer == NLAYER - 1

        def edge_grp(g, _):
            pltpu.sync_copy(fr_hbm.at[s, pl.ds(g * IG, IG)], fr_g)
            pltpu.sync_copy(to_hbm.at[s, pl.ds(g * IG, IG)], to_g)

            def edge_blk(b, _):
                gds = []
                for u in range(NB):
                    gds.append(pltpu.async_copy(
                        s_sh.at[fr_g.at[b * NB + u]], bufs[u], gsems[u]))
                sds = []
                for u in range(NB):
                    gds[u].wait()
                    sds.append(pltpu.async_copy(
                        bufs[u], r_sh.at[to_g.at[b * NB + u]], ssems[u],
                        add=True))
                for d in sds:
                    d.wait()
                return 0
            lax.fori_loop(0, IG // NB, edge_blk, 0)
            return 0
        lax.fori_loop(0, NGROUP, edge_grp, 0)

        plsc.subcore_barrier()

        def ew_blk(j, _):
            r0 = base + j * EWC
            pltpu.sync_copy(r_sh.at[pl.ds(r0, EWC)], ew_r)
            if layer == 0:
                pltpu.sync_copy(ew_r, out_hbm.at[c, pl.ds(r0, EWC)])
            else:
                pltpu.sync_copy(out_hbm.at[c, pl.ds(r0, EWC)], ew_a)
            if last:
                pltpu.sync_copy(emb_hbm.at[c, pl.ds(r0, EWC)],
                                buf0.at[pl.ds(0, EWC)])

            def row(i, _):
                row_l = j * EWC + i
                for cv in range(4):
                    sl = pl.ds(cv * 16, 16)
                    rv = ew_r[i, sl]
                    if last:
                        b = _bcast16(dis_v, row_l)
                        acc = ew_a[i, sl] + rv
                        sbuf[i, sl] = 0.25 * buf0[i, sl] + 0.25 * b * acc
                    else:
                        b2 = _bcast16(dis2_v, row_l)
                        if layer > 0:
                            ew_a[i, sl] = ew_a[i, sl] + rv
                        sbuf[i, sl] = b2 * rv
                return 0
            lax.fori_loop(0, EWC, row, 0)

            if last:
                pltpu.sync_copy(sbuf, out_hbm.at[c, pl.ds(r0, EWC)])
            else:
                if layer > 0:
                    pltpu.sync_copy(ew_a, out_hbm.at[c, pl.ds(r0, EWC)])
                pltpu.sync_copy(sbuf, s_sh.at[pl.ds(r0, EWC)])

                def zf(i, _):
                    for cv in range(4):
                        sbuf[i, pl.ds(cv * 16, 16)] = (
                            jnp.zeros((16,), jnp.float32))
                    return 0
                lax.fori_loop(0, EWC, zf, 0)
                pltpu.sync_copy(sbuf, r_sh.at[pl.ds(r0, EWC)])
            return 0
        lax.fori_loop(0, n_ew, ew_blk, 0)

        if not last:
            plsc.subcore_barrier()


_sc_call = pl.kernel(
    _sc_body,
    out_type=jax.ShapeDtypeStruct((NCORE, NNODE, DH), jnp.float32),
    mesh=plsc.VectorSubcoreMesh(
        core_axis_name="c", subcore_axis_name="s",
        num_cores=NCORE, num_subcores=NSUB),
    scratch_types=[
        pltpu.VMEM_SHARED((NPAD2, DH), jnp.float32),
        pltpu.VMEM_SHARED((NPAD2, DH), jnp.float32),
        pltpu.VMEM_SHARED((NPAD2,), jnp.float32),
        pltpu.VMEM((IG, CH), jnp.int32),
        pltpu.VMEM((IG, CH), jnp.int32),
        pltpu.VMEM((CH, DH), jnp.float32),
        pltpu.VMEM((CH, DH), jnp.float32),
        pltpu.VMEM((CH, DH), jnp.float32),
        pltpu.VMEM((CH, DH), jnp.float32),
        pltpu.VMEM((CH, DH), jnp.float32),
        pltpu.VMEM((EWC, DH), jnp.float32),
        pltpu.VMEM((EWC, DH), jnp.float32),
        pltpu.VMEM((EWC, DH), jnp.float32),
        pltpu.VMEM((RPW,), jnp.float32),
        pltpu.VMEM((RPW,), jnp.float32),
        pltpu.VMEM((CH,), jnp.float32),
        pltpu.SemaphoreType.DMA,
        pltpu.SemaphoreType.DMA,
        pltpu.SemaphoreType.DMA,
        pltpu.SemaphoreType.DMA,
        pltpu.SemaphoreType.DMA,
        pltpu.SemaphoreType.DMA,
        pltpu.SemaphoreType.DMA,
        pltpu.SemaphoreType.DMA,
        pltpu.SemaphoreType.DMA,
        pltpu.SemaphoreType.DMA,
    ],
    compiler_params=pltpu.CompilerParams(
        needs_layout_passes=False, use_tc_tiling_on_sc=False),
)


@jax.jit
def kernel(edge_index, edge_attrs, emb_weight):
    del edge_attrs
    npad = EPAD - NEDGE
    padidx = (jnp.arange(npad, dtype=jnp.int32) % PADROWS) + NNODE
    fr3 = jnp.concatenate([edge_index[0], padidx]).reshape(NSUB, NCHUNK, CH)
    to3 = jnp.concatenate([edge_index[1], padidx]).reshape(NSUB, NCHUNK, CH)
    emb2 = emb_weight.reshape(NNODE, NCORE, DH).transpose(1, 0, 2)
    out2 = _sc_call(fr3, to3, emb2)
    out = out2.transpose(1, 0, 2).reshape(NNODE, DDIM)
    return (emb_weight, out)


# speedup vs baseline: 1.1728x; 1.0379x over previous
import jax
import jax.numpy as jnp
from jax import lax
from jax.experimental import pallas as pl
from jax.experimental.pallas import tpu as pltpu
from jax.experimental.pallas import tpu_sc as plsc

NNODE = 10000
DDIM = 128
NEDGE = 320000
NLAYER = 3

NCORE = 2
NSUB = 16
DH = DDIM // NCORE
CH = 64
NB = 5
IG = 20
NGROUP = 16
NCHUNK = 320
EPT = NCHUNK * CH
EPAD = NSUB * EPT
PADROWS = 8
RPW = 640
NPAD2 = NSUB * RPW
EWC = 80


def _bcast16(ref, idx):
    return plsc.load_gather(ref, [jnp.full((16,), idx, jnp.int32)])


def _sc_body(fr_hbm, to_hbm, emb_hbm, out_hbm,
             s_sh, r_sh, deg_sh,
             fr_g, to_g, buf0, buf1, buf2, buf3, buf4, ew_r, ew_a, sbuf,
             dis_v, dis2_v, ones_v,
             gsem0, gsem1, gsem2, gsem3, gsem4,
             ssem0, ssem1, ssem2, ssem3, ssem4):
    bufs = (buf0, buf1, buf2, buf3, buf4)
    gsems = (gsem0, gsem1, gsem2, gsem3, gsem4)
    ssems = (ssem0, ssem1, ssem2, ssem3, ssem4)
    c = lax.axis_index("c")
    s = lax.axis_index("s")
    base = s * RPW
    nrows = jnp.minimum(RPW, jnp.maximum(0, NNODE - base))
    n_ew = nrows // EWC

    def fill_ones(k, _):
        ones_v[pl.ds(k * 16, 16)] = jnp.ones((16,), jnp.float32)
        return 0
    lax.fori_loop(0, CH // 16, fill_ones, 0)

    def zfill(i, _):
        for cv in range(4):
            sbuf[i, pl.ds(cv * 16, 16)] = jnp.zeros((16,), jnp.float32)
        return 0
    lax.fori_loop(0, EWC, zfill, 0)

    def zero_deg(k, _):
        pltpu.sync_copy(sbuf.at[0], deg_sh.at[pl.ds(base + k * DH, DH)])
        return 0
    lax.fori_loop(0, RPW // DH, zero_deg, 0)

    def zero_r(j, _):
        pltpu.sync_copy(sbuf, r_sh.at[pl.ds(base + j * EWC, EWC)])
        return 0
    lax.fori_loop(0, RPW // EWC, zero_r, 0)

    def zero_spad(j, _):
        pltpu.sync_copy(sbuf, s_sh.at[pl.ds(base + j * EWC, EWC)])
        return 0
    lax.fori_loop(n_ew, RPW // EWC, zero_spad, 0)

    plsc.subcore_barrier()

    def deg_grp(g, _):
        pltpu.sync_copy(to_hbm.at[s, pl.ds(g * IG, IG)], to_g)

        def deg_blk(b, _):
            descs = []
            for u in range(NB):
                descs.append(pltpu.async_copy(
                    ones_v, deg_sh.at[to_g.at[b * NB + u]], ssems[u],
                    add=True))
            for d in descs:
                d.wait()
            return 0
        lax.fori_loop(0, IG // NB, deg_blk, 0)
        return 0
    lax.fori_loop(0, NGROUP, deg_grp, 0)

    plsc.subcore_barrier()

    pltpu.sync_copy(deg_sh.at[pl.ds(base, RPW)], dis_v)

    def newton(k, _):
        sl = pl.ds(k * 16, 16)
        d = dis_v[sl]
        i = lax.bitcast_convert_type(d, jnp.int32)
        i = jnp.int32(0x5F3759DF) - lax.shift_right_logical(i, 1)
        y = lax.bitcast_convert_type(i, jnp.float32)
        for _ in range(3):
            y = y * (1.5 - 0.5 * d * y * y)
        y = jnp.where(d > 0.0, y, 0.0)
        dis_v[sl] = y
        dis2_v[sl] = y * y
        return 0
    lax.fori_loop(0, RPW // 16, newton, 0)

    def s0_blk(j, _):
        r0 = base + j * EWC
        pltpu.sync_copy(emb_hbm.at[c, pl.ds(r0, EWC)], ew_r)

        def row(i, _):
            b = _bcast16(dis_v, j * EWC + i)
            for cv in range(4):
                sl = pl.ds(cv * 16, 16)
                sbuf[i, sl] = b * ew_r[i, sl]
            return 0
        lax.fori_loop(0, EWC, row, 0)
        pltpu.sync_copy(sbuf, s_sh.at[pl.ds(r0, EWC)])
        return 0
    lax.fori_loop(0, n_ew, s0_blk, 0)

    plsc.subcore_barrier()

    for layer in range(NLAYER):
        last = layer == NLAYER - 1

        def edge_grp(g, _):
            pltpu.sync_copy(fr_hbm.at[s, pl.ds(g * IG, IG)], fr_g)
            pltpu.sync_copy(to_hbm.at[s, pl.ds(g * IG, IG)], to_g)

            def edge_blk(b, _):
                gds = []
                for u in range(NB):
                    gds.append(pltpu.async_copy(
                        s_sh.at[fr_g.at[b * NB + u]], bufs[u], gsems[u]))
                sds = []
                for u in range(NB):
                    gds[u].wait()
                    sds.append(pltpu.async_copy(
                        bufs[u], r_sh.at[to_g.at[b * NB + u]], ssems[u],
                        add=True))
                for d in sds:
                    d.wait()
                return 0
            lax.fori_loop(0, IG // NB, edge_blk, 0)
            return 0
        lax.fori_loop(0, NGROUP, edge_grp, 0)

        plsc.subcore_barrier()

        def ew_blk(j, _):
            r0 = base + j * EWC
            pltpu.sync_copy(r_sh.at[pl.ds(r0, EWC)], ew_r)
            if layer == 0:
                pltpu.sync_copy(ew_r, out_hbm.at[c, pl.ds(r0, EWC)])
            else:
                pltpu.sync_copy(out_hbm.at[c, pl.ds(r0, EWC)], ew_a)
            if last:
                pltpu.sync_copy(emb_hbm.at[c, pl.ds(r0, EWC)],
                                buf0.at[pl.ds(0, EWC)])

            def row(i, _):
                row_l = j * EWC + i
                for cv in range(4):
                    sl = pl.ds(cv * 16, 16)
                    rv = ew_r[i, sl]
                    if last:
                        b = _bcast16(dis_v, row_l)
                        acc = ew_a[i, sl] + rv
                        sbuf[i, sl] = 0.25 * buf0[i, sl] + 0.25 * b * acc
                    else:
                        b2 = _bcast16(dis2_v, row_l)
                        if layer > 0:
                            ew_a[i, sl] = ew_a[i, sl] + rv
                        sbuf[i, sl] = b2 * rv
                return 0
            lax.fori_loop(0, EWC, row, 0)

            if last:
                pltpu.sync_copy(sbuf, out_hbm.at[c, pl.ds(r0, EWC)])
            else:
                if layer > 0:
                    pltpu.sync_copy(ew_a, out_hbm.at[c, pl.ds(r0, EWC)])
                pltpu.sync_copy(sbuf, s_sh.at[pl.ds(r0, EWC)])

                def zf(i, _):
                    for cv in range(4):
                        sbuf[i, pl.ds(cv * 16, 16)] = (
                            jnp.zeros((16,), jnp.float32))
                    return 0
                lax.fori_loop(0, EWC, zf, 0)
                pltpu.sync_copy(sbuf, r_sh.at[pl.ds(r0, EWC)])
            return 0
        lax.fori_loop(0, n_ew, ew_blk, 0)

        if not last:
            plsc.subcore_barrier()


_sc_call = pl.kernel(
    _sc_body,
    out_type=jax.ShapeDtypeStruct((NCORE, NNODE, DH), jnp.float32),
    mesh=plsc.VectorSubcoreMesh(
        core_axis_name="c", subcore_axis_name="s",
        num_cores=NCORE, num_subcores=NSUB),
    scratch_types=[
        pltpu.VMEM_SHARED((NPAD2, DH), jnp.float32),
        pltpu.VMEM_SHARED((NPAD2, DH), jnp.float32),
        pltpu.VMEM_SHARED((NPAD2,), jnp.float32),
        pltpu.VMEM((IG, CH), jnp.int32),
        pltpu.VMEM((IG, CH), jnp.int32),
        pltpu.VMEM((CH, DH), jnp.float32),
        pltpu.VMEM((CH, DH), jnp.float32),
        pltpu.VMEM((CH, DH), jnp.float32),
        pltpu.VMEM((CH, DH), jnp.float32),
        pltpu.VMEM((CH, DH), jnp.float32),
        pltpu.VMEM((EWC, DH), jnp.float32),
        pltpu.VMEM((EWC, DH), jnp.float32),
        pltpu.VMEM((EWC, DH), jnp.float32),
        pltpu.VMEM((RPW,), jnp.float32),
        pltpu.VMEM((RPW,), jnp.float32),
        pltpu.VMEM((CH,), jnp.float32),
        pltpu.SemaphoreType.DMA,
        pltpu.SemaphoreType.DMA,
        pltpu.SemaphoreType.DMA,
        pltpu.SemaphoreType.DMA,
        pltpu.SemaphoreType.DMA,
        pltpu.SemaphoreType.DMA,
        pltpu.SemaphoreType.DMA,
        pltpu.SemaphoreType.DMA,
        pltpu.SemaphoreType.DMA,
        pltpu.SemaphoreType.DMA,
    ],
    compiler_params=pltpu.CompilerParams(
        needs_layout_passes=False, use_tc_tiling_on_sc=False),
)


@jax.jit
def kernel(edge_index, edge_attrs, emb_weight):
    del edge_attrs
    npad = EPAD - NEDGE
    padidx = (jnp.arange(npad, dtype=jnp.int32) % PADROWS) + NNODE
    fr3 = jnp.concatenate([edge_index[0], padidx]).reshape(NSUB, NCHUNK, CH)
    to3 = jnp.concatenate([edge_index[1], padidx]).reshape(NSUB, NCHUNK, CH)
    emb2 = emb_weight.reshape(NNODE, NCORE, DH).transpose(1, 0, 2)
    out2 = _sc_call(fr3, to3, emb2)
    out = out2.transpose(1, 0, 2).reshape(NNODE, DDIM)
    return (emb_weight, out)


# speedup vs baseline: 1.2856x; 1.0961x over previous
import jax
import jax.numpy as jnp
from jax import lax
from jax.experimental import pallas as pl
from jax.experimental.pallas import tpu as pltpu
from jax.experimental.pallas import tpu_sc as plsc

NNODE = 10000
DDIM = 128
NEDGE = 320000
NLAYER = 3

NCORE = 2
NSUB = 16
DH = DDIM // NCORE
CH = 64
NB = 4
IG = 16
NGROUP = 20
NCHUNK = 320
EPT = NCHUNK * CH
EPAD = NSUB * EPT
PADROWS = 8
RPW = 640
NPAD2 = NSUB * RPW
EWC = 80


def _bcast16(ref, idx):
    return plsc.load_gather(ref, [jnp.full((16,), idx, jnp.int32)])


def _sc_body(fr_hbm, to_hbm, emb_hbm, out_hbm,
             s_sh, r_sh, deg_sh,
             fr_g, to_g, buf0, buf1, buf2, buf3, ew_r, ew_a, sbuf,
             dis_v, dis2_v, ones_v,
             gsem0, gsem1, gsem2, gsem3,
             ssem0, ssem1, ssem2, ssem3):
    bufs = (buf0, buf1, buf2, buf3)
    gsems = (gsem0, gsem1, gsem2, gsem3)
    ssems = (ssem0, ssem1, ssem2, ssem3)
    c = lax.axis_index("c")
    s = lax.axis_index("s")
    base = s * RPW
    nrows = jnp.minimum(RPW, jnp.maximum(0, NNODE - base))
    n_ew = nrows // EWC

    def fill_ones(k, _):
        ones_v[pl.ds(k * 16, 16)] = jnp.ones((16,), jnp.float32)
        return 0
    lax.fori_loop(0, CH // 16, fill_ones, 0)

    def zfill(i, _):
        for cv in range(4):
            sbuf[i, pl.ds(cv * 16, 16)] = jnp.zeros((16,), jnp.float32)
        return 0
    lax.fori_loop(0, EWC, zfill, 0)

    def zero_deg(k, _):
        pltpu.sync_copy(sbuf.at[0], deg_sh.at[pl.ds(base + k * DH, DH)])
        return 0
    lax.fori_loop(0, RPW // DH, zero_deg, 0)

    def zero_r(j, _):
        pltpu.sync_copy(sbuf, r_sh.at[pl.ds(base + j * EWC, EWC)])
        return 0
    lax.fori_loop(0, RPW // EWC, zero_r, 0)

    def zero_spad(j, _):
        pltpu.sync_copy(sbuf, s_sh.at[pl.ds(base + j * EWC, EWC)])
        return 0
    lax.fori_loop(n_ew, RPW // EWC, zero_spad, 0)

    plsc.subcore_barrier()

    def deg_grp(g, _):
        pltpu.sync_copy(to_hbm.at[s, pl.ds(g * IG, IG)], to_g)
        for u in range(IG):
            pltpu.async_copy(ones_v, deg_sh.at[to_g.at[u]], ssems[0],
                             add=True)
        return 0
    lax.fori_loop(0, NGROUP, deg_grp, 0)

    def deg_drain(k, _):
        pltpu.make_async_copy(ones_v, deg_sh.at[to_g.at[0]], ssems[0]).wait()
        return 0
    lax.fori_loop(0, NCHUNK, deg_drain, 0)

    plsc.subcore_barrier()

    pltpu.sync_copy(deg_sh.at[pl.ds(base, RPW)], dis_v)

    def newton(k, _):
        sl = pl.ds(k * 16, 16)
        d = dis_v[sl]
        i = lax.bitcast_convert_type(d, jnp.int32)
        i = jnp.int32(0x5F3759DF) - lax.shift_right_logical(i, 1)
        y = lax.bitcast_convert_type(i, jnp.float32)
        for _ in range(3):
            y = y * (1.5 - 0.5 * d * y * y)
        y = jnp.where(d > 0.0, y, 0.0)
        dis_v[sl] = y
        dis2_v[sl] = y * y
        return 0
    lax.fori_loop(0, RPW // 16, newton, 0)

    def s0_blk(j, _):
        r0 = base + j * EWC
        pltpu.sync_copy(emb_hbm.at[c, pl.ds(r0, EWC)], ew_r)

        def row(i, _):
            b = _bcast16(dis_v, j * EWC + i)
            for cv in range(4):
                sl = pl.ds(cv * 16, 16)
                sbuf[i, sl] = b * ew_r[i, sl]
            return 0
        lax.fori_loop(0, EWC, row, 0)
        pltpu.sync_copy(sbuf, s_sh.at[pl.ds(r0, EWC)])
        return 0
    lax.fori_loop(0, n_ew, s0_blk, 0)

    plsc.subcore_barrier()

    for layer in range(NLAYER):
        last = layer == NLAYER - 1

        def edge_grp(g, _):
            pltpu.sync_copy(fr_hbm.at[s, pl.ds(g * IG, IG)], fr_g)
            pltpu.sync_copy(to_hbm.at[s, pl.ds(g * IG, IG)], to_g)

            def edge_blk(b, _):
                gb = g * (IG // NB) + b

                @pl.when(gb > 0)
                def _wait_prev():
                    for u in range(NB):
                        pltpu.make_async_copy(
                            bufs[u], r_sh.at[to_g.at[u]], ssems[u]).wait()

                gds = []
                for u in range(NB):
                    gds.append(pltpu.async_copy(
                        s_sh.at[fr_g.at[b * NB + u]], bufs[u], gsems[u]))
                for u in range(NB):
                    gds[u].wait()
                    pltpu.async_copy(
                        bufs[u], r_sh.at[to_g.at[b * NB + u]], ssems[u],
                        add=True)
                return 0
            lax.fori_loop(0, IG // NB, edge_blk, 0)
            return 0
        lax.fori_loop(0, NGROUP, edge_grp, 0)

        for u in range(NB):
            pltpu.make_async_copy(bufs[u], r_sh.at[to_g.at[u]], ssems[u]).wait()

        plsc.subcore_barrier()

        def ew_blk(j, _):
            r0 = base + j * EWC
            pltpu.sync_copy(r_sh.at[pl.ds(r0, EWC)], ew_r)
            if layer == 0:
                pltpu.sync_copy(ew_r, out_hbm.at[c, pl.ds(r0, EWC)])
            else:
                pltpu.sync_copy(out_hbm.at[c, pl.ds(r0, EWC)], ew_a)
            if last:
                pltpu.sync_copy(emb_hbm.at[c, pl.ds(r0, EWC)],
                                buf0.at[pl.ds(0, EWC)])

            def row(i, _):
                row_l = j * EWC + i
                for cv in range(4):
                    sl = pl.ds(cv * 16, 16)
                    rv = ew_r[i, sl]
                    if last:
                        b = _bcast16(dis_v, row_l)
                        acc = ew_a[i, sl] + rv
                        sbuf[i, sl] = 0.25 * buf0[i, sl] + 0.25 * b * acc
                    else:
                        b2 = _bcast16(dis2_v, row_l)
                        if layer > 0:
                            ew_a[i, sl] = ew_a[i, sl] + rv
                        sbuf[i, sl] = b2 * rv
                return 0
            lax.fori_loop(0, EWC, row, 0)

            if last:
                pltpu.sync_copy(sbuf, out_hbm.at[c, pl.ds(r0, EWC)])
            else:
                if layer > 0:
                    pltpu.sync_copy(ew_a, out_hbm.at[c, pl.ds(r0, EWC)])
                pltpu.sync_copy(sbuf, s_sh.at[pl.ds(r0, EWC)])

                def zf(i, _):
                    for cv in range(4):
                        sbuf[i, pl.ds(cv * 16, 16)] = (
                            jnp.zeros((16,), jnp.float32))
                    return 0
                lax.fori_loop(0, EWC, zf, 0)
                pltpu.sync_copy(sbuf, r_sh.at[pl.ds(r0, EWC)])
            return 0
        lax.fori_loop(0, n_ew, ew_blk, 0)

        if not last:
            plsc.subcore_barrier()


_sc_call = pl.kernel(
    _sc_body,
    out_type=jax.ShapeDtypeStruct((NCORE, NNODE, DH), jnp.float32),
    mesh=plsc.VectorSubcoreMesh(
        core_axis_name="c", subcore_axis_name="s",
        num_cores=NCORE, num_subcores=NSUB),
    scratch_types=[
        pltpu.VMEM_SHARED((NPAD2, DH), jnp.float32),
        pltpu.VMEM_SHARED((NPAD2, DH), jnp.float32),
        pltpu.VMEM_SHARED((NPAD2,), jnp.float32),
        pltpu.VMEM((IG, CH), jnp.int32),
        pltpu.VMEM((IG, CH), jnp.int32),
        pltpu.VMEM((CH, DH), jnp.float32),
        pltpu.VMEM((CH, DH), jnp.float32),
        pltpu.VMEM((CH, DH), jnp.float32),
        pltpu.VMEM((CH, DH), jnp.float32),
        pltpu.VMEM((EWC, DH), jnp.float32),
        pltpu.VMEM((EWC, DH), jnp.float32),
        pltpu.VMEM((EWC, DH), jnp.float32),
        pltpu.VMEM((RPW,), jnp.float32),
        pltpu.VMEM((RPW,), jnp.float32),
        pltpu.VMEM((CH,), jnp.float32),
        pltpu.SemaphoreType.DMA,
        pltpu.SemaphoreType.DMA,
        pltpu.SemaphoreType.DMA,
        pltpu.SemaphoreType.DMA,
        pltpu.SemaphoreType.DMA,
        pltpu.SemaphoreType.DMA,
        pltpu.SemaphoreType.DMA,
        pltpu.SemaphoreType.DMA,
    ],
    compiler_params=pltpu.CompilerParams(
        needs_layout_passes=False, use_tc_tiling_on_sc=False),
)


@jax.jit
def kernel(edge_index, edge_attrs, emb_weight):
    del edge_attrs
    npad = EPAD - NEDGE
    padidx = (jnp.arange(npad, dtype=jnp.int32) % PADROWS) + NNODE
    fr3 = jnp.concatenate([edge_index[0], padidx]).reshape(NSUB, NCHUNK, CH)
    to3 = jnp.concatenate([edge_index[1], padidx]).reshape(NSUB, NCHUNK, CH)
    emb2 = emb_weight.reshape(NNODE, NCORE, DH).transpose(1, 0, 2)
    out2 = _sc_call(fr3, to3, emb2)
    out = out2.transpose(1, 0, 2).reshape(NNODE, DDIM)
    return (emb_weight, out)


# speedup vs baseline: 1.3350x; 1.0385x over previous
import jax
import jax.numpy as jnp
from jax import lax
from jax.experimental import pallas as pl
from jax.experimental.pallas import tpu as pltpu
from jax.experimental.pallas import tpu_sc as plsc

NNODE = 10000
DDIM = 128
NEDGE = 320000
NLAYER = 3

NCORE = 2
NSUB = 16
DH = DDIM // NCORE
CH = 64
NB = 4
IG = 32
NGROUP = 10
NCHUNK = 320
EPT = NCHUNK * CH
EPAD = NSUB * EPT
PADROWS = 8
RPW = 640
NPAD2 = NSUB * RPW
EWC = 80


def _bcast16(ref, idx):
    return plsc.load_gather(ref, [jnp.full((16,), idx, jnp.int32)])


def _sc_body(fr_hbm, to_hbm, emb_hbm, out_hbm,
             s_sh, r_sh, deg_sh,
             fr_g, to_g, buf0, buf1, buf2, buf3, ew_r, ew_a, sbuf,
             dis_v, dis2_v, ones_v,
             gsem0, gsem1, gsem2, gsem3,
             ssem0, ssem1, ssem2, ssem3):
    bufs = (buf0, buf1, buf2, buf3)
    gsems = (gsem0, gsem1, gsem2, gsem3)
    ssems = (ssem0, ssem1, ssem2, ssem3)
    c = lax.axis_index("c")
    s = lax.axis_index("s")
    base = s * RPW
    nrows = jnp.minimum(RPW, jnp.maximum(0, NNODE - base))
    n_ew = nrows // EWC

    def fill_ones(k, _):
        ones_v[pl.ds(k * 16, 16)] = jnp.ones((16,), jnp.float32)
        return 0
    lax.fori_loop(0, CH // 16, fill_ones, 0)

    def zfill(i, _):
        for cv in range(4):
            sbuf[i, pl.ds(cv * 16, 16)] = jnp.zeros((16,), jnp.float32)
        return 0
    lax.fori_loop(0, EWC, zfill, 0)

    def zero_deg(k, _):
        pltpu.sync_copy(sbuf.at[0], deg_sh.at[pl.ds(base + k * DH, DH)])
        return 0
    lax.fori_loop(0, RPW // DH, zero_deg, 0)

    def zero_r(j, _):
        pltpu.sync_copy(sbuf, r_sh.at[pl.ds(base + j * EWC, EWC)])
        return 0
    lax.fori_loop(0, RPW // EWC, zero_r, 0)

    def zero_spad(j, _):
        pltpu.sync_copy(sbuf, s_sh.at[pl.ds(base + j * EWC, EWC)])
        return 0
    lax.fori_loop(n_ew, RPW // EWC, zero_spad, 0)

    plsc.subcore_barrier()

    def deg_grp(g, _):
        pltpu.sync_copy(to_hbm.at[s, pl.ds(g * IG, IG)], to_g)
        for u in range(IG):
            pltpu.async_copy(ones_v, deg_sh.at[to_g.at[u]], ssems[0],
                             add=True)
        return 0
    lax.fori_loop(0, NGROUP, deg_grp, 0)

    def deg_drain(k, _):
        pltpu.make_async_copy(ones_v, deg_sh.at[to_g.at[0]], ssems[0]).wait()
        return 0
    lax.fori_loop(0, NCHUNK, deg_drain, 0)

    plsc.subcore_barrier()

    pltpu.sync_copy(deg_sh.at[pl.ds(base, RPW)], dis_v)

    def newton(k, _):
        sl = pl.ds(k * 16, 16)
        d = dis_v[sl]
        i = lax.bitcast_convert_type(d, jnp.int32)
        i = jnp.int32(0x5F3759DF) - lax.shift_right_logical(i, 1)
        y = lax.bitcast_convert_type(i, jnp.float32)
        for _ in range(3):
            y = y * (1.5 - 0.5 * d * y * y)
        y = jnp.where(d > 0.0, y, 0.0)
        dis_v[sl] = y
        dis2_v[sl] = y * y
        return 0
    lax.fori_loop(0, RPW // 16, newton, 0)

    def s0_blk(j, _):
        r0 = base + j * EWC
        pltpu.sync_copy(emb_hbm.at[c, pl.ds(r0, EWC)], ew_r)

        def row(i, _):
            b = _bcast16(dis_v, j * EWC + i)
            for cv in range(4):
                sl = pl.ds(cv * 16, 16)
                sbuf[i, sl] = b * ew_r[i, sl]
            return 0
        lax.fori_loop(0, EWC, row, 0)
        pltpu.sync_copy(sbuf, s_sh.at[pl.ds(r0, EWC)])
        return 0
    lax.fori_loop(0, n_ew, s0_blk, 0)

    plsc.subcore_barrier()

    for layer in range(NLAYER):
        last = layer == NLAYER - 1

        def edge_grp(g, _):
            pltpu.sync_copy(fr_hbm.at[s, pl.ds(g * IG, IG)], fr_g)
            pltpu.sync_copy(to_hbm.at[s, pl.ds(g * IG, IG)], to_g)

            def edge_blk(b, _):
                gb = g * (IG // NB) + b

                @pl.when(gb > 0)
                def _wait_prev():
                    for u in range(NB):
                        pltpu.make_async_copy(
                            bufs[u], r_sh.at[to_g.at[u]], ssems[u]).wait()

                gds = []
                for u in range(NB):
                    gds.append(pltpu.async_copy(
                        s_sh.at[fr_g.at[b * NB + u]], bufs[u], gsems[u]))
                for u in range(NB):
                    gds[u].wait()
                    pltpu.async_copy(
                        bufs[u], r_sh.at[to_g.at[b * NB + u]], ssems[u],
                        add=True)
                return 0
            lax.fori_loop(0, IG // NB, edge_blk, 0)
            return 0
        lax.fori_loop(0, NGROUP, edge_grp, 0)

        for u in range(NB):
            pltpu.make_async_copy(bufs[u], r_sh.at[to_g.at[u]], ssems[u]).wait()

        plsc.subcore_barrier()

        def ew_blk(j, _):
            r0 = base + j * EWC
            pltpu.sync_copy(r_sh.at[pl.ds(r0, EWC)], ew_r)
            if layer == 0:
                pltpu.sync_copy(ew_r, out_hbm.at[c, pl.ds(r0, EWC)])
            else:
                pltpu.sync_copy(out_hbm.at[c, pl.ds(r0, EWC)], ew_a)
            if last:
                pltpu.sync_copy(emb_hbm.at[c, pl.ds(r0, EWC)],
                                buf0.at[pl.ds(0, EWC)])

            def row(i, _):
                row_l = j * EWC + i
                for cv in range(4):
                    sl = pl.ds(cv * 16, 16)
                    rv = ew_r[i, sl]
                    if last:
                        b = _bcast16(dis_v, row_l)
                        acc = ew_a[i, sl] + rv
                        sbuf[i, sl] = 0.25 * buf0[i, sl] + 0.25 * b * acc
                    else:
                        b2 = _bcast16(dis2_v, row_l)
                        if layer > 0:
                            ew_a[i, sl] = ew_a[i, sl] + rv
                        sbuf[i, sl] = b2 * rv
                return 0
            lax.fori_loop(0, EWC, row, 0)

            if last:
                pltpu.sync_copy(sbuf, out_hbm.at[c, pl.ds(r0, EWC)])
            else:
                if layer > 0:
                    pltpu.sync_copy(ew_a, out_hbm.at[c, pl.ds(r0, EWC)])
                pltpu.sync_copy(sbuf, s_sh.at[pl.ds(r0, EWC)])

                def zf(i, _):
                    for cv in range(4):
                        sbuf[i, pl.ds(cv * 16, 16)] = (
                            jnp.zeros((16,), jnp.float32))
                    return 0
                lax.fori_loop(0, EWC, zf, 0)
                pltpu.sync_copy(sbuf, r_sh.at[pl.ds(r0, EWC)])
            return 0
        lax.fori_loop(0, n_ew, ew_blk, 0)

        if not last:
            plsc.subcore_barrier()


_sc_call = pl.kernel(
    _sc_body,
    out_type=jax.ShapeDtypeStruct((NCORE, NNODE, DH), jnp.float32),
    mesh=plsc.VectorSubcoreMesh(
        core_axis_name="c", subcore_axis_name="s",
        num_cores=NCORE, num_subcores=NSUB),
    scratch_types=[
        pltpu.VMEM_SHARED((NPAD2, DH), jnp.float32),
        pltpu.VMEM_SHARED((NPAD2, DH), jnp.float32),
        pltpu.VMEM_SHARED((NPAD2,), jnp.float32),
        pltpu.VMEM((IG, CH), jnp.int32),
        pltpu.VMEM((IG, CH), jnp.int32),
        pltpu.VMEM((CH, DH), jnp.float32),
        pltpu.VMEM((CH, DH), jnp.float32),
        pltpu.VMEM((CH, DH), jnp.float32),
        pltpu.VMEM((CH, DH), jnp.float32),
        pltpu.VMEM((EWC, DH), jnp.float32),
        pltpu.VMEM((EWC, DH), jnp.float32),
        pltpu.VMEM((EWC, DH), jnp.float32),
        pltpu.VMEM((RPW,), jnp.float32),
        pltpu.VMEM((RPW,), jnp.float32),
        pltpu.VMEM((CH,), jnp.float32),
        pltpu.SemaphoreType.DMA,
        pltpu.SemaphoreType.DMA,
        pltpu.SemaphoreType.DMA,
        pltpu.SemaphoreType.DMA,
        pltpu.SemaphoreType.DMA,
        pltpu.SemaphoreType.DMA,
        pltpu.SemaphoreType.DMA,
        pltpu.SemaphoreType.DMA,
    ],
    compiler_params=pltpu.CompilerParams(
        needs_layout_passes=False, use_tc_tiling_on_sc=False),
)


@jax.jit
def kernel(edge_index, edge_attrs, emb_weight):
    del edge_attrs
    npad = EPAD - NEDGE
    padidx = (jnp.arange(npad, dtype=jnp.int32) % PADROWS) + NNODE
    fr3 = jnp.concatenate([edge_index[0], padidx]).reshape(NSUB, NCHUNK, CH)
    to3 = jnp.concatenate([edge_index[1], padidx]).reshape(NSUB, NCHUNK, CH)
    emb2 = emb_weight.reshape(NNODE, NCORE, DH).transpose(1, 0, 2)
    out2 = _sc_call(fr3, to3, emb2)
    out = out2.transpose(1, 0, 2).reshape(NNODE, DDIM)
    return (emb_weight, out)
